# baseline (speedup 1.0000x reference)
import jax
import jax.numpy as jnp
from jax.experimental import pallas as pl
from jax.experimental.pallas import tpu as pltpu

N_TOKENS = 16384
IN_DIM = 1024
HIDDEN = 256
NUM_EXPERTS = 16

BM = 512


def _router_body(x_ref, w1_ref, b1_ref, w2_ref, b2_ref, probs_ref, eid_ref):
    x = x_ref[...]
    h = jnp.dot(x, w1_ref[...], preferred_element_type=jnp.float32)
    h = jnp.maximum(h + b1_ref[...], 0.0)
    logits = jnp.dot(h, w2_ref[...], preferred_element_type=jnp.float32)
    logits = logits + b2_ref[...]
    m = jnp.max(logits, axis=-1, keepdims=True)
    e = jnp.exp(logits - m)
    probs_ref[...] = e / jnp.sum(e, axis=-1, keepdims=True)
    eid_ref[...] = jnp.argmax(logits, axis=-1, keepdims=True).astype(jnp.int32)


def kernel(patch_feat, W1, b1, W2, b2):
    b1_2d = b1.reshape(1, HIDDEN)
    b2_2d = b2.reshape(1, NUM_EXPERTS)
    grid = (N_TOKENS // BM,)
    probs, eid = pl.pallas_call(
        _router_body,
        grid=grid,
        in_specs=[
            pl.BlockSpec((BM, IN_DIM), lambda i: (i, 0)),
            pl.BlockSpec((IN_DIM, HIDDEN), lambda i: (0, 0)),
            pl.BlockSpec((1, HIDDEN), lambda i: (0, 0)),
            pl.BlockSpec((HIDDEN, NUM_EXPERTS), lambda i: (0, 0)),
            pl.BlockSpec((1, NUM_EXPERTS), lambda i: (0, 0)),
        ],
        out_specs=[
            pl.BlockSpec((BM, NUM_EXPERTS), lambda i: (i, 0)),
            pl.BlockSpec((BM, 1), lambda i: (i, 0)),
        ],
        out_shape=[
            jax.ShapeDtypeStruct((N_TOKENS, NUM_EXPERTS), jnp.float32),
            jax.ShapeDtypeStruct((N_TOKENS, 1), jnp.int32),
        ],
        compiler_params=pltpu.CompilerParams(
            dimension_semantics=("parallel",),
        ),
    )(patch_feat, W1, b1_2d, W2, b2_2d)
    return probs, eid.reshape(N_TOKENS)


# speedup vs baseline: 1.3799x; 1.3799x over previous
import jax
import jax.numpy as jnp
from jax.experimental import pallas as pl
from jax.experimental.pallas import tpu as pltpu

N_TOKENS = 16384
IN_DIM = 1024
HIDDEN = 256
NUM_EXPERTS = 16

BM = 2048


def _router_body(x_ref, w1_ref, b1_ref, w2_ref, b2_ref, probs_ref, eid_ref):
    x = x_ref[...]
    h = jnp.dot(x, w1_ref[...], preferred_element_type=jnp.float32)
    h = jnp.maximum(h + b1_ref[...], 0.0)
    logits = jnp.dot(h, w2_ref[...], preferred_element_type=jnp.float32)
    logits = logits + b2_ref[...]
    m = jnp.max(logits, axis=-1, keepdims=True)
    e = jnp.exp(logits - m)
    probs_ref[...] = e / jnp.sum(e, axis=-1, keepdims=True)
    eid_ref[...] = jnp.argmax(logits, axis=-1, keepdims=True).astype(jnp.int32)


def kernel(patch_feat, W1, b1, W2, b2):
    b1_2d = b1.reshape(1, HIDDEN)
    b2_2d = b2.reshape(1, NUM_EXPERTS)
    grid = (N_TOKENS // BM,)
    probs, eid = pl.pallas_call(
        _router_body,
        grid=grid,
        in_specs=[
            pl.BlockSpec((BM, IN_DIM), lambda i: (i, 0)),
            pl.BlockSpec((IN_DIM, HIDDEN), lambda i: (0, 0)),
            pl.BlockSpec((1, HIDDEN), lambda i: (0, 0)),
            pl.BlockSpec((HIDDEN, NUM_EXPERTS), lambda i: (0, 0)),
            pl.BlockSpec((1, NUM_EXPERTS), lambda i: (0, 0)),
        ],
        out_specs=[
            pl.BlockSpec((BM, NUM_EXPERTS), lambda i: (i, 0)),
            pl.BlockSpec((BM, 1), lambda i: (i, 0)),
        ],
        out_shape=[
            jax.ShapeDtypeStruct((N_TOKENS, NUM_EXPERTS), jnp.float32),
            jax.ShapeDtypeStruct((N_TOKENS, 1), jnp.int32),
        ],
        compiler_params=pltpu.CompilerParams(
            dimension_semantics=("parallel",),
        ),
    )(patch_feat, W1, b1_2d, W2, b2_2d)
    return probs, eid.reshape(N_TOKENS)


# speedup vs baseline: 1.4146x; 1.0252x over previous
import jax
import jax.numpy as jnp
from jax.experimental import pallas as pl
from jax.experimental.pallas import tpu as pltpu

N_TOKENS = 16384
IN_DIM = 1024
HIDDEN = 256
NUM_EXPERTS = 16

BM = 4096


def _router_body(x_ref, w1_ref, b1_ref, w2_ref, b2_ref, probs_ref, eid_ref):
    x = x_ref[...]
    h = jnp.dot(x, w1_ref[...], preferred_element_type=jnp.float32)
    h = jnp.maximum(h + b1_ref[...], 0.0)
    logits = jnp.dot(h, w2_ref[...], preferred_element_type=jnp.float32)
    logits = logits + b2_ref[...]
    m = jnp.max(logits, axis=-1, keepdims=True)
    e = jnp.exp(logits - m)
    probs_ref[...] = e / jnp.sum(e, axis=-1, keepdims=True)
    eid_ref[...] = jnp.argmax(logits, axis=-1, keepdims=True).astype(jnp.int32)


def kernel(patch_feat, W1, b1, W2, b2):
    b1_2d = b1.reshape(1, HIDDEN)
    b2_2d = b2.reshape(1, NUM_EXPERTS)
    grid = (N_TOKENS // BM,)
    probs, eid = pl.pallas_call(
        _router_body,
        grid=grid,
        in_specs=[
            pl.BlockSpec((BM, IN_DIM), lambda i: (i, 0)),
            pl.BlockSpec((IN_DIM, HIDDEN), lambda i: (0, 0)),
            pl.BlockSpec((1, HIDDEN), lambda i: (0, 0)),
            pl.BlockSpec((HIDDEN, NUM_EXPERTS), lambda i: (0, 0)),
            pl.BlockSpec((1, NUM_EXPERTS), lambda i: (0, 0)),
        ],
        out_specs=[
            pl.BlockSpec((BM, NUM_EXPERTS), lambda i: (i, 0)),
            pl.BlockSpec((BM, 1), lambda i: (i, 0)),
        ],
        out_shape=[
            jax.ShapeDtypeStruct((N_TOKENS, NUM_EXPERTS), jnp.float32),
            jax.ShapeDtypeStruct((N_TOKENS, 1), jnp.int32),
        ],
        compiler_params=pltpu.CompilerParams(
            dimension_semantics=("parallel",),
        ),
    )(patch_feat, W1, b1_2d, W2, b2_2d)
    return probs, eid.reshape(N_TOKENS)
